# baseline (speedup 1.0000x reference)
import functools

import jax
import jax.numpy as jnp
from jax import lax
from jax.experimental import pallas as pl
from jax.experimental.pallas import tpu as pltpu
from jax.experimental.pallas import tpu_sc as plsc

N = 15
E = 210
EL = E + N
D0 = 480
D1 = 1920
G2 = 3
T2 = D1 // G2

RSPLIT = 896
RSC = D1 - RSPLIT
NWORK = 32
RPW = RSC // NWORK
GTC = RSPLIT // 128
ROWG = 16


def _dot(a, b):
    return jax.lax.dot_general(a, b, (((1,), (0,)), ((), ())),
                               preferred_element_type=jnp.float32)


def _dot_t(a, b):
    return jax.lax.dot_general(a, b, (((1,), (1,)), ((), ())),
                               preferred_element_type=jnp.float32)


def _embed_body(f_ref, we1_ref, be1_ref, we2_ref, be2_ref, x1_ref, x0_scr):
    g = pl.program_id(0)

    @pl.when(g == 0)
    def _():
        x0_scr[...] = jnp.maximum(
            _dot_t(f_ref[...], we1_ref[...]) + be1_ref[...], 0.0)

    x1_ref[...] = jnp.maximum(
        _dot_t(x0_scr[...], we2_ref[...]) + be2_ref[0], 0.0)


_sc_mesh = plsc.VectorSubcoreMesh(core_axis_name="c", subcore_axis_name="s")


@functools.partial(
    pl.kernel,
    mesh=_sc_mesh,
    out_type=jax.ShapeDtypeStruct((RSC,), jnp.float32),
    scratch_types=[
        pltpu.VMEM((D1,), jnp.float32),
        pltpu.VMEM((RPW, D1), jnp.float32),
        pltpu.VMEM((RPW,), jnp.float32),
        pltpu.VMEM((RPW,), jnp.float32),
        pltpu.VMEM((ROWG, 16), jnp.float32),
    ],
)
def _sc_gemv(x1_hbm, w_hbm, b_hbm, out_hbm, x1_v, w_v, b_v, o_v, t_v):
    wid = lax.axis_index("s") * 2 + lax.axis_index("c")
    base = RSPLIT + wid * RPW
    pltpu.sync_copy(x1_hbm, x1_v)
    pltpu.sync_copy(w_hbm.at[pl.ds(base, RPW), :], w_v)
    pltpu.sync_copy(b_hbm.at[pl.ds(base, RPW)], b_v)

    lanes = lax.iota(jnp.int32, 16)

    def _row_group(gi, carry):
        r0 = gi * ROWG

        def _chunk(c, accs):
            x1c = x1_v[pl.ds(c * 16, 16)]
            return tuple(accs[j] + w_v[r0 + j, pl.ds(c * 16, 16)] * x1c
                         for j in range(ROWG))

        accs = lax.fori_loop(
            0, D1 // 16, _chunk,
            tuple(jnp.zeros((16,), jnp.float32) for _ in range(ROWG)))
        out = jnp.zeros((16,), jnp.float32)
        for j in range(ROWG):
            a = accs[j]
            for sh in (1, 2, 4, 8):
                a = a + a.at[lanes ^ sh].get(mode="promise_in_bounds")
            out = jnp.where(lanes == j, a, out)
        o_v[pl.ds(r0, ROWG)] = out + b_v[pl.ds(r0, ROWG)]
        return carry

    lax.fori_loop(0, RPW // ROWG, _row_group, 0)
    pltpu.sync_copy(o_v, out_hbm.at[pl.ds(wid * RPW, RPW)])


def _tc_gemv_body(x1_ref, we3_ref, be3_ref, out_ref):
    out_ref[...] = _dot_t(x1_ref[...], we3_ref[...]) + be3_ref[0]


def _tail_body(lo_ref, hi_ref, s_ref, d_ref, tgt_ref,
               wc1_ref, bc1_ref, wc2_ref, bc2_ref,
               watt_ref, wfc_ref, bfc_ref,
               loss_ref, preds_ref):
    lo = lo_ref[...]
    hi = hi_ref[...]
    x2 = jnp.concatenate(
        [lo[:, 128 * n:128 * (n + 1)] for n in range(RSPLIT // 128)]
        + [hi[:, 128 * n:128 * (n + 1)] for n in range(RSC // 128)],
        axis=0)

    s_ids = s_ref[...]
    d_ids = d_ref[...]
    nodes = jax.lax.broadcasted_iota(jnp.int32, (N, EL), 0)
    s_oh = (nodes == s_ids).astype(jnp.float32)
    d_oh = (nodes == d_ids).astype(jnp.float32)
    adj = _dot_t(d_oh, s_oh)
    ones_row = jnp.ones((1, N), jnp.float32)
    ones_col = jnp.ones((N, 1), jnp.float32)
    deg_col = _dot(adj, ones_col)
    deg_row = _dot_t(ones_row, adj)
    dis_col = jnp.where(deg_col > 0, jax.lax.rsqrt(deg_col), 0.0)
    dis_row = jnp.where(deg_row > 0, jax.lax.rsqrt(deg_row), 0.0)
    a_norm = adj * dis_col * dis_row

    h1 = _dot(a_norm, _dot_t(x2, wc1_ref[...])) + bc1_ref[...]
    h1 = jnp.maximum(h1, 0.0)
    h2 = _dot(a_norm, _dot_t(h1, wc2_ref[...])) + bc2_ref[...]

    gc = _dot(ones_row, _dot(h2, watt_ref[...])) * (1.0 / N)
    tg = jnp.tanh(gc)
    scores = jax.nn.sigmoid(_dot_t(h2, tg))
    rep = jnp.sum(h2 * scores, axis=0, keepdims=True)
    logits = _dot_t(rep, wfc_ref[...]) + bfc_ref[...]

    tgt = tgt_ref[...]
    idx3 = jax.lax.broadcasted_iota(jnp.int32, (1, 3), 1)
    tmax = jnp.max(tgt, axis=1, keepdims=True)
    label = jnp.min(jnp.where(tgt >= tmax, idx3, 3), axis=1, keepdims=True)
    m = jnp.max(logits, axis=1, keepdims=True)
    ex = jnp.exp(logits - m)
    sex = jnp.sum(ex, axis=1, keepdims=True)
    logsm = logits - m - jnp.log(sex)
    loss_ref[...] = -jnp.sum(jnp.where(idx3 == label, logsm, 0.0),
                             axis=1, keepdims=True)
    preds_ref[...] = ex / sex


def kernel(features_1, edge_index_1, target, W_e1, b_e1, W_e2, b_e2,
           W_e3, b_e3, W_c1, b_c1, W_c2, b_c2, W_att, W_fc, b_fc):
    loop = jnp.arange(N, dtype=edge_index_1.dtype)
    s = jnp.concatenate([edge_index_1[0], loop]).reshape(1, EL)
    d = jnp.concatenate([edge_index_1[1], loop]).reshape(1, EL)
    f = features_1.reshape(1, N)

    x1 = pl.pallas_call(
        _embed_body,
        grid=(G2,),
        in_specs=[
            pl.BlockSpec((1, N), lambda g: (0, 0)),
            pl.BlockSpec((D0, N), lambda g: (0, 0)),
            pl.BlockSpec((1, D0), lambda g: (0, 0)),
            pl.BlockSpec((T2, D0), lambda g: (g, 0)),
            pl.BlockSpec((1, 1, T2), lambda g: (g, 0, 0)),
        ],
        out_specs=pl.BlockSpec((1, T2), lambda g: (0, g)),
        out_shape=jax.ShapeDtypeStruct((1, D1), jnp.float32),
        scratch_shapes=[pltpu.VMEM((1, D0), jnp.float32)],
    )(f, W_e1, b_e1.reshape(1, D0), W_e2, b_e2.reshape(G2, 1, T2))

    x2_hi = _sc_gemv(x1.reshape(D1), W_e3, b_e3)

    x2_lo = pl.pallas_call(
        _tc_gemv_body,
        grid=(GTC,),
        in_specs=[
            pl.BlockSpec((1, D1), lambda g: (0, 0)),
            pl.BlockSpec((128, D1), lambda g: (g, 0)),
            pl.BlockSpec((1, 1, 128), lambda g: (g, 0, 0)),
        ],
        out_specs=pl.BlockSpec((1, 128), lambda g: (0, g)),
        out_shape=jax.ShapeDtypeStruct((1, RSPLIT), jnp.float32),
    )(x1, W_e3, b_e3[:RSPLIT].reshape(GTC, 1, 128))

    def _full(shape):
        return pl.BlockSpec(shape, lambda: (0,) * len(shape))

    loss2d, preds2d = pl.pallas_call(
        _tail_body,
        in_specs=[
            _full((1, RSPLIT)),
            _full((1, RSC)),
            _full((1, EL)),
            _full((1, EL)),
            _full((1, 3)),
            _full((128, 128)),
            _full((1, 128)),
            _full((64, 128)),
            _full((1, 64)),
            _full((64, 64)),
            _full((3, 64)),
            _full((1, 3)),
        ],
        out_specs=(_full((1, 1)), _full((1, 3))),
        out_shape=(jax.ShapeDtypeStruct((1, 1), jnp.float32),
                   jax.ShapeDtypeStruct((1, 3), jnp.float32)),
    )(x2_lo, x2_hi.reshape(1, RSC), s, d, target.reshape(1, 3),
      W_c1, b_c1.reshape(1, 128), W_c2, b_c2.reshape(1, 64),
      W_att, W_fc, b_fc.reshape(1, 3))
    return (loss2d[0, 0], preds2d[0])


# speedup vs baseline: 1.7541x; 1.7541x over previous
import jax
import jax.numpy as jnp
from jax.experimental import pallas as pl
from jax.experimental.pallas import tpu as pltpu

N = 15
E = 210
EL = E + N
D0 = 480
D1 = 1920
G2 = 5
T2 = D1 // G2
G3 = 3
T3 = D1 // G3
GRID = G2 + G3


def _dot(a, b):
    return jax.lax.dot_general(a, b, (((1,), (0,)), ((), ())),
                               preferred_element_type=jnp.float32)


def _dot_t(a, b):
    return jax.lax.dot_general(a, b, (((1,), (1,)), ((), ())),
                               preferred_element_type=jnp.float32)


def _body(f_ref, s_ref, d_ref, tgt_ref,
          we1_ref, be1_ref, we2_ref, be2_ref, we3_ref, be3_ref,
          wc1_ref, bc1_ref, wc2_ref, bc2_ref,
          watt_ref, wfc_ref, bfc_ref,
          loss_ref, preds_ref,
          x0_scr, x1_scr):
    g = pl.program_id(0)

    @pl.when(g == 0)
    def _embed1():
        x0_scr[...] = jnp.maximum(
            _dot_t(f_ref[...], we1_ref[...]) + be1_ref[...], 0.0)

    @pl.when(g < G2)
    def _embed2_tile():
        x1_scr[0, pl.ds(g * T2, T2)] = jnp.maximum(
            _dot_t(x0_scr[...], we2_ref[...]) + be2_ref[0], 0.0)[0]

    @pl.when(g >= G2)
    def _embed3_tile():
        x1_scr[0, pl.ds(D1 + (g - G2) * T3, T3)] = (
            _dot_t(x1_scr[0:1, 0:D1], we3_ref[...]) + be3_ref[0])[0]

    @pl.when(g == GRID - 1)
    def _graph_tail():
        x2f = x1_scr[0:1, D1:2 * D1]
        x2 = jnp.concatenate(
            [x2f[:, 128 * n:128 * (n + 1)] for n in range(N)], axis=0)

        s_ids = s_ref[...]
        d_ids = d_ref[...]
        nodes = jax.lax.broadcasted_iota(jnp.int32, (N, EL), 0)
        s_oh = (nodes == s_ids).astype(jnp.float32)
        d_oh = (nodes == d_ids).astype(jnp.float32)
        adj = _dot_t(d_oh, s_oh)
        ones_row = jnp.ones((1, N), jnp.float32)
        ones_col = jnp.ones((N, 1), jnp.float32)
        deg_col = _dot(adj, ones_col)
        deg_row = _dot_t(ones_row, adj)
        dis_col = jnp.where(deg_col > 0, jax.lax.rsqrt(deg_col), 0.0)
        dis_row = jnp.where(deg_row > 0, jax.lax.rsqrt(deg_row), 0.0)
        a_norm = adj * dis_col * dis_row

        h1 = _dot(a_norm, _dot_t(x2, wc1_ref[...])) + bc1_ref[...]
        h1 = jnp.maximum(h1, 0.0)
        h2 = _dot(a_norm, _dot_t(h1, wc2_ref[...])) + bc2_ref[...]

        gc = _dot(ones_row, _dot(h2, watt_ref[...])) * (1.0 / N)
        tg = jnp.tanh(gc)
        scores = jax.nn.sigmoid(_dot_t(h2, tg))
        rep = jnp.sum(h2 * scores, axis=0, keepdims=True)
        logits = _dot_t(rep, wfc_ref[...]) + bfc_ref[...]

        tgt = tgt_ref[...]
        idx3 = jax.lax.broadcasted_iota(jnp.int32, (1, 3), 1)
        tmax = jnp.max(tgt, axis=1, keepdims=True)
        label = jnp.min(jnp.where(tgt >= tmax, idx3, 3), axis=1,
                        keepdims=True)
        m = jnp.max(logits, axis=1, keepdims=True)
        ex = jnp.exp(logits - m)
        sex = jnp.sum(ex, axis=1, keepdims=True)
        logsm = logits - m - jnp.log(sex)
        loss_ref[...] = -jnp.sum(jnp.where(idx3 == label, logsm, 0.0),
                                 axis=1, keepdims=True)
        preds_ref[...] = ex / sex


def _full(shape):
    return pl.BlockSpec(shape, lambda g: (0,) * len(shape))


def kernel(features_1, edge_index_1, target, W_e1, b_e1, W_e2, b_e2,
           W_e3, b_e3, W_c1, b_c1, W_c2, b_c2, W_att, W_fc, b_fc):
    loop = jnp.arange(N, dtype=edge_index_1.dtype)
    s = jnp.concatenate([edge_index_1[0], loop]).reshape(1, EL)
    d = jnp.concatenate([edge_index_1[1], loop]).reshape(1, EL)
    f = features_1.reshape(1, N)
    args = (f, s, d, target.reshape(1, 3),
            W_e1, b_e1.reshape(1, -1), W_e2, b_e2.reshape(G2, 1, T2),
            W_e3, b_e3.reshape(G3, 1, T3),
            W_c1, b_c1.reshape(1, -1), W_c2, b_c2.reshape(1, -1),
            W_att, W_fc, b_fc.reshape(1, -1))
    in_specs = [
        _full((1, N)),
        _full((1, EL)),
        _full((1, EL)),
        _full((1, 3)),
        _full((D0, N)),
        _full((1, D0)),
        pl.BlockSpec((T2, D0), lambda g: (jnp.minimum(g, G2 - 1), 0)),
        pl.BlockSpec((1, 1, T2), lambda g: (jnp.minimum(g, G2 - 1), 0, 0)),
        pl.BlockSpec((T3, D1),
                     lambda g: (jnp.clip(g - G2, 0, G3 - 1), 0)),
        pl.BlockSpec((1, 1, T3),
                     lambda g: (jnp.clip(g - G2, 0, G3 - 1), 0, 0)),
        _full((128, 128)),
        _full((1, 128)),
        _full((64, 128)),
        _full((1, 64)),
        _full((64, 64)),
        _full((3, 64)),
        _full((1, 3)),
    ]
    loss2d, preds2d = pl.pallas_call(
        _body,
        grid=(GRID,),
        in_specs=in_specs,
        out_specs=(_full((1, 1)), _full((1, 3))),
        out_shape=(jax.ShapeDtypeStruct((1, 1), jnp.float32),
                   jax.ShapeDtypeStruct((1, 3), jnp.float32)),
        scratch_shapes=[
            pltpu.VMEM((1, D0), jnp.float32),
            pltpu.VMEM((1, 2 * D1), jnp.float32),
        ],
    )(*args)
    return (loss2d[0, 0], preds2d[0])


# speedup vs baseline: 1.8077x; 1.0306x over previous
import jax
import jax.numpy as jnp
from jax.experimental import pallas as pl
from jax.experimental.pallas import tpu as pltpu

N = 15
E = 210
EL = E + N
D0 = 480
D1 = 1920
G2 = 3
T2 = D1 // G2
G3 = 1
T3 = D1 // G3
GRID = G2 + G3


def _dot(a, b):
    return jax.lax.dot_general(a, b, (((1,), (0,)), ((), ())),
                               preferred_element_type=jnp.float32)


def _dot_t(a, b):
    return jax.lax.dot_general(a, b, (((1,), (1,)), ((), ())),
                               preferred_element_type=jnp.float32)


def _body(f_ref, s_ref, d_ref, tgt_ref,
          we1_ref, be1_ref, we2_ref, be2_ref, we3_ref, be3_ref,
          wc1_ref, bc1_ref, wc2_ref, bc2_ref,
          watt_ref, wfc_ref, bfc_ref,
          loss_ref, preds_ref,
          x0_scr, x1_scr):
    g = pl.program_id(0)

    @pl.when(g == 0)
    def _embed1():
        x0_scr[...] = jnp.maximum(
            _dot_t(f_ref[...], we1_ref[...]) + be1_ref[...], 0.0)

    @pl.when(g < G2)
    def _embed2_tile():
        x1_scr[0, pl.ds(g * T2, T2)] = jnp.maximum(
            _dot_t(x0_scr[...], we2_ref[...]) + be2_ref[0], 0.0)[0]

    @pl.when(g >= G2)
    def _embed3_tile():
        x1_scr[0, pl.ds(D1 + (g - G2) * T3, T3)] = (
            _dot_t(x1_scr[0:1, 0:D1], we3_ref[...]) + be3_ref[0])[0]

    @pl.when(g == GRID - 1)
    def _graph_tail():
        x2f = x1_scr[0:1, D1:2 * D1]
        x2 = jnp.concatenate(
            [x2f[:, 128 * n:128 * (n + 1)] for n in range(N)], axis=0)

        s_ids = s_ref[...]
        d_ids = d_ref[...]
        nodes = jax.lax.broadcasted_iota(jnp.int32, (N, EL), 0)
        s_oh = (nodes == s_ids).astype(jnp.float32)
        d_oh = (nodes == d_ids).astype(jnp.float32)
        adj = _dot_t(d_oh, s_oh)
        ones_row = jnp.ones((1, N), jnp.float32)
        ones_col = jnp.ones((N, 1), jnp.float32)
        deg_col = _dot(adj, ones_col)
        deg_row = _dot_t(ones_row, adj)
        dis_col = jnp.where(deg_col > 0, jax.lax.rsqrt(deg_col), 0.0)
        dis_row = jnp.where(deg_row > 0, jax.lax.rsqrt(deg_row), 0.0)
        a_norm = adj * dis_col * dis_row

        h1 = _dot(a_norm, _dot_t(x2, wc1_ref[...])) + bc1_ref[...]
        h1 = jnp.maximum(h1, 0.0)
        h2 = _dot(a_norm, _dot_t(h1, wc2_ref[...])) + bc2_ref[...]

        gc = _dot(ones_row, _dot(h2, watt_ref[...])) * (1.0 / N)
        tg = jnp.tanh(gc)
        scores = jax.nn.sigmoid(_dot_t(h2, tg))
        rep = jnp.sum(h2 * scores, axis=0, keepdims=True)
        logits = _dot_t(rep, wfc_ref[...]) + bfc_ref[...]

        tgt = tgt_ref[...]
        idx3 = jax.lax.broadcasted_iota(jnp.int32, (1, 3), 1)
        tmax = jnp.max(tgt, axis=1, keepdims=True)
        label = jnp.min(jnp.where(tgt >= tmax, idx3, 3), axis=1,
                        keepdims=True)
        m = jnp.max(logits, axis=1, keepdims=True)
        ex = jnp.exp(logits - m)
        sex = jnp.sum(ex, axis=1, keepdims=True)
        logsm = logits - m - jnp.log(sex)
        loss_ref[...] = -jnp.sum(jnp.where(idx3 == label, logsm, 0.0),
                                 axis=1, keepdims=True)
        preds_ref[...] = ex / sex


def _full(shape):
    return pl.BlockSpec(shape, lambda g: (0,) * len(shape))


def kernel(features_1, edge_index_1, target, W_e1, b_e1, W_e2, b_e2,
           W_e3, b_e3, W_c1, b_c1, W_c2, b_c2, W_att, W_fc, b_fc):
    loop = jnp.arange(N, dtype=edge_index_1.dtype)
    s = jnp.concatenate([edge_index_1[0], loop]).reshape(1, EL)
    d = jnp.concatenate([edge_index_1[1], loop]).reshape(1, EL)
    f = features_1.reshape(1, N)
    args = (f, s, d, target.reshape(1, 3),
            W_e1, b_e1.reshape(1, -1), W_e2, b_e2.reshape(G2, 1, T2),
            W_e3, b_e3.reshape(G3, 1, T3),
            W_c1, b_c1.reshape(1, -1), W_c2, b_c2.reshape(1, -1),
            W_att, W_fc, b_fc.reshape(1, -1))
    in_specs = [
        _full((1, N)),
        _full((1, EL)),
        _full((1, EL)),
        _full((1, 3)),
        _full((D0, N)),
        _full((1, D0)),
        pl.BlockSpec((T2, D0), lambda g: (jnp.minimum(g, G2 - 1), 0)),
        pl.BlockSpec((1, 1, T2), lambda g: (jnp.minimum(g, G2 - 1), 0, 0)),
        pl.BlockSpec((T3, D1),
                     lambda g: (jnp.clip(g - G2, 0, G3 - 1), 0)),
        pl.BlockSpec((1, 1, T3),
                     lambda g: (jnp.clip(g - G2, 0, G3 - 1), 0, 0)),
        _full((128, 128)),
        _full((1, 128)),
        _full((64, 128)),
        _full((1, 64)),
        _full((64, 64)),
        _full((3, 64)),
        _full((1, 3)),
    ]
    loss2d, preds2d = pl.pallas_call(
        _body,
        grid=(GRID,),
        in_specs=in_specs,
        out_specs=(_full((1, 1)), _full((1, 3))),
        out_shape=(jax.ShapeDtypeStruct((1, 1), jnp.float32),
                   jax.ShapeDtypeStruct((1, 3), jnp.float32)),
        scratch_shapes=[
            pltpu.VMEM((1, D0), jnp.float32),
            pltpu.VMEM((1, 2 * D1), jnp.float32),
        ],
    )(*args)
    return (loss2d[0, 0], preds2d[0])


# speedup vs baseline: 1.8096x; 1.0010x over previous
import jax
import jax.numpy as jnp
from jax.experimental import pallas as pl
from jax.experimental.pallas import tpu as pltpu

N = 15
E = 210
EL = E + N
D0 = 480
D1 = 1920
G2 = 3
T2 = D1 // G2
G3 = 3
T3 = D1 // G3
GRID = G2 + G3


def _dot(a, b):
    return jax.lax.dot_general(a, b, (((1,), (0,)), ((), ())),
                               preferred_element_type=jnp.float32)


def _dot_t(a, b):
    return jax.lax.dot_general(a, b, (((1,), (1,)), ((), ())),
                               preferred_element_type=jnp.float32)


def _body(f_ref, s_ref, d_ref, tgt_ref,
          we1_ref, be1_ref, we2_ref, be2_ref, we3_ref, be3_ref,
          wc1_ref, bc1_ref, wc2_ref, bc2_ref,
          watt_ref, wfc_ref, bfc_ref,
          loss_ref, preds_ref,
          x0_scr, x1_scr, an_scr):
    g = pl.program_id(0)

    @pl.when(g == 0)
    def _embed1():
        x0_scr[...] = jnp.maximum(
            _dot_t(f_ref[...], we1_ref[...]) + be1_ref[...], 0.0)

        s_ids = s_ref[...]
        d_ids = d_ref[...]
        nodes = jax.lax.broadcasted_iota(jnp.int32, (N, EL), 0)
        s_oh = (nodes == s_ids).astype(jnp.float32)
        d_oh = (nodes == d_ids).astype(jnp.float32)
        adj = _dot_t(d_oh, s_oh)
        ones_row = jnp.ones((1, N), jnp.float32)
        ones_col = jnp.ones((N, 1), jnp.float32)
        deg_col = _dot(adj, ones_col)
        deg_row = _dot_t(ones_row, adj)
        dis_col = jnp.where(deg_col > 0, jax.lax.rsqrt(deg_col), 0.0)
        dis_row = jnp.where(deg_row > 0, jax.lax.rsqrt(deg_row), 0.0)
        an_scr[...] = adj * dis_col * dis_row

    @pl.when(g < G2)
    def _embed2_tile():
        x1_scr[0, pl.ds(g * T2, T2)] = jnp.maximum(
            _dot_t(x0_scr[...], we2_ref[...]) + be2_ref[0], 0.0)[0]

    @pl.when(g >= G2)
    def _embed3_tile():
        x1_scr[0, pl.ds(D1 + (g - G2) * T3, T3)] = (
            _dot_t(x1_scr[0:1, 0:D1], we3_ref[...]) + be3_ref[0])[0]

    @pl.when(g == GRID - 1)
    def _graph_tail():
        x2f = x1_scr[0:1, D1:2 * D1]
        x2 = jnp.concatenate(
            [x2f[:, 128 * n:128 * (n + 1)] for n in range(N)], axis=0)

        a_norm = an_scr[...]
        ones_row = jnp.ones((1, N), jnp.float32)

        h1 = _dot(a_norm, _dot_t(x2, wc1_ref[...])) + bc1_ref[...]
        h1 = jnp.maximum(h1, 0.0)
        h2 = _dot(a_norm, _dot_t(h1, wc2_ref[...])) + bc2_ref[...]

        gc = _dot(ones_row, _dot(h2, watt_ref[...])) * (1.0 / N)
        tg = jnp.tanh(gc)
        scores = jax.nn.sigmoid(_dot_t(h2, tg))
        rep = jnp.sum(h2 * scores, axis=0, keepdims=True)
        logits = _dot_t(rep, wfc_ref[...]) + bfc_ref[...]

        tgt = tgt_ref[...]
        idx3 = jax.lax.broadcasted_iota(jnp.int32, (1, 3), 1)
        tmax = jnp.max(tgt, axis=1, keepdims=True)
        label = jnp.min(jnp.where(tgt >= tmax, idx3, 3), axis=1,
                        keepdims=True)
        m = jnp.max(logits, axis=1, keepdims=True)
        ex = jnp.exp(logits - m)
        sex = jnp.sum(ex, axis=1, keepdims=True)
        logsm = logits - m - jnp.log(sex)
        loss_ref[...] = -jnp.sum(jnp.where(idx3 == label, logsm, 0.0),
                                 axis=1, keepdims=True)
        preds_ref[...] = ex / sex


def _full(shape):
    return pl.BlockSpec(shape, lambda g: (0,) * len(shape))


def kernel(features_1, edge_index_1, target, W_e1, b_e1, W_e2, b_e2,
           W_e3, b_e3, W_c1, b_c1, W_c2, b_c2, W_att, W_fc, b_fc):
    loop = jnp.arange(N, dtype=edge_index_1.dtype)
    s = jnp.concatenate([edge_index_1[0], loop]).reshape(1, EL)
    d = jnp.concatenate([edge_index_1[1], loop]).reshape(1, EL)
    f = features_1.reshape(1, N)
    args = (f, s, d, target.reshape(1, 3),
            W_e1, b_e1.reshape(1, -1), W_e2, b_e2.reshape(G2, 1, T2),
            W_e3, b_e3.reshape(G3, 1, T3),
            W_c1, b_c1.reshape(1, -1), W_c2, b_c2.reshape(1, -1),
            W_att, W_fc, b_fc.reshape(1, -1))
    in_specs = [
        _full((1, N)),
        _full((1, EL)),
        _full((1, EL)),
        _full((1, 3)),
        _full((D0, N)),
        _full((1, D0)),
        pl.BlockSpec((T2, D0), lambda g: (jnp.minimum(g, G2 - 1), 0)),
        pl.BlockSpec((1, 1, T2), lambda g: (jnp.minimum(g, G2 - 1), 0, 0)),
        pl.BlockSpec((T3, D1),
                     lambda g: (jnp.clip(g - G2, 0, G3 - 1), 0)),
        pl.BlockSpec((1, 1, T3),
                     lambda g: (jnp.clip(g - G2, 0, G3 - 1), 0, 0)),
        _full((128, 128)),
        _full((1, 128)),
        _full((64, 128)),
        _full((1, 64)),
        _full((64, 64)),
        _full((3, 64)),
        _full((1, 3)),
    ]
    loss2d, preds2d = pl.pallas_call(
        _body,
        grid=(GRID,),
        in_specs=in_specs,
        out_specs=(_full((1, 1)), _full((1, 3))),
        out_shape=(jax.ShapeDtypeStruct((1, 1), jnp.float32),
                   jax.ShapeDtypeStruct((1, 3), jnp.float32)),
        scratch_shapes=[
            pltpu.VMEM((1, D0), jnp.float32),
            pltpu.VMEM((1, 2 * D1), jnp.float32),
            pltpu.VMEM((N, N), jnp.float32),
        ],
    )(*args)
    return (loss2d[0, 0], preds2d[0])
